# baseline (speedup 1.0000x reference)
import functools

import jax
import jax.numpy as jnp
from jax import lax
from jax.experimental import pallas as pl
from jax.experimental.pallas import tpu as pltpu
from jax.experimental.pallas import tpu_sc as plsc

NW = 32
KC = 128
EPS = 1e-12
RB = 16


def _gather_body(idx_hbm, table_hbm, out_hbm, idx_v, rows0, rows1,
                 sem0, sem1):
    nchunk, kc = idx_v.shape
    wid = lax.axis_index("s") * 2 + lax.axis_index("c")
    base = wid * nchunk * kc
    rows = (rows0, rows1)
    sems = (sem0, sem1)
    pltpu.sync_copy(idx_hbm.at[wid], idx_v)

    def start(g, b):
        pltpu.async_copy(table_hbm.at[idx_v.at[g]], rows[b], sems[b])

    def wait(b):
        pltpu.make_async_copy(table_hbm.at[idx_v.at[0]], rows[b],
                              sems[b]).wait()

    start(0, 0)

    def chunk_pair(g2, _):
        for b in range(2):
            g = g2 * 2 + b

            @pl.when(g + 1 < nchunk)
            def _():
                start(g + 1, 1 - b)

            wait(b)
            pltpu.sync_copy(rows[b], out_hbm.at[pl.ds(base + g * kc, kc)])
        return ()

    lax.fori_loop(0, nchunk // 2, chunk_pair, (), unroll=False)

    if nchunk % 2 == 1:
        wait(0)
        pltpu.sync_copy(rows[0],
                        out_hbm.at[pl.ds(base + (nchunk - 1) * kc, kc)])


def _sc_gather(idx, table):
    nw, nchunk, kc = idx.shape
    t = nw * nchunk * kc
    word_dim = table.shape[1]
    mesh = plsc.VectorSubcoreMesh(core_axis_name="c", subcore_axis_name="s",
                                  num_cores=2, num_subcores=16)
    return pl.kernel(
        _gather_body,
        out_type=jax.ShapeDtypeStruct((t, word_dim), jnp.float32),
        mesh=mesh,
        scratch_types=[
            pltpu.VMEM((nchunk, kc), jnp.int32),
            pltpu.VMEM((kc, word_dim), jnp.float32),
            pltpu.VMEM((kc, word_dim), jnp.float32),
            pltpu.SemaphoreType.DMA,
            pltpu.SemaphoreType.DMA,
        ],
    )(idx, table)


def _pad_body(src_ref, out_ref):
    rb, wd = src_ref.shape
    wdp = out_ref.shape[-1]
    out_ref[...] = jnp.pad(src_ref[...], ((0, 0), (0, wdp - wd)))


def _tc_pad(table, wdp):
    v, wd = table.shape
    rb = 2000
    return pl.pallas_call(
        _pad_body,
        out_shape=jax.ShapeDtypeStruct((v, wdp), jnp.float32),
        grid=(v // rb,),
        in_specs=[pl.BlockSpec((rb, wd), lambda i: (i, 0))],
        out_specs=pl.BlockSpec((rb, wdp), lambda i: (i, 0)),
    )(table)


def _ln_compute(words_ref, chars_ref, pos_ref, gamma_ref, beta_ref, out_ref):
    wd = out_ref.shape[-1] - chars_ref.shape[-1]
    x = jnp.concatenate([words_ref[..., :wd], chars_ref[...]], axis=-1)
    x = x + pos_ref[...][None, :, :]
    mu = jnp.mean(x, axis=-1, keepdims=True)
    var = jnp.mean(jnp.square(x - mu), axis=-1, keepdims=True)
    y = (x - mu) * lax.rsqrt(var + EPS)
    out_ref[...] = y * gamma_ref[...][None] + beta_ref[...][None]


def _ln_body(words_ref, chars_ref, pos_ref, gamma_ref, beta_ref, out_ref):
    _ln_compute(words_ref, chars_ref, pos_ref, gamma_ref, beta_ref, out_ref)


def _ln_body_alias(words_ref, chars_ref, pos_ref, gamma_ref, beta_ref,
                   prev_ref, out_ref):
    del prev_ref
    _ln_compute(words_ref, chars_ref, pos_ref, gamma_ref, beta_ref, out_ref)


def _tc_ln_half(words, chars, pos, gamma, beta, off, out_prev):
    bh, l, wdp = words.shape
    cd = chars.shape[-1]
    h = pos.shape[-1]
    bfull = chars.shape[0]
    grid = (bh // RB,)
    in_specs = [
        pl.BlockSpec((RB, l, wdp), lambda i: (i, 0, 0)),
        pl.BlockSpec((RB, l, cd), lambda i: (i + off, 0, 0)),
        pl.BlockSpec((l, h), lambda i: (0, 0)),
        pl.BlockSpec((1, h), lambda i: (0, 0)),
        pl.BlockSpec((1, h), lambda i: (0, 0)),
    ]
    args = [words, chars, pos, gamma, beta]
    kwargs = {}
    body = _ln_body
    if out_prev is not None:
        in_specs.append(pl.BlockSpec((RB, l, h), lambda i: (0, 0, 0)))
        args.append(out_prev)
        kwargs["input_output_aliases"] = {5: 0}
        body = _ln_body_alias
    return pl.pallas_call(
        body,
        out_shape=jax.ShapeDtypeStruct((bfull, l, h), jnp.float32),
        grid=grid,
        in_specs=in_specs,
        out_specs=pl.BlockSpec((RB, l, h), lambda i: (i + off, 0, 0)),
        **kwargs,
    )(*args)


@jax.jit
def kernel(word_ids, chars_embeddings, word_table, pos_table, gamma, beta):
    b, l = word_ids.shape
    idx = word_ids.reshape(2, NW, -1, KC).astype(jnp.int32)
    table_p = _tc_pad(word_table, 384)
    pos = pos_table[:l]
    g2 = gamma.reshape(1, -1)
    b2 = beta.reshape(1, -1)
    w0 = _sc_gather(idx[0], table_p).reshape(b // 2, l, 384)
    w1 = _sc_gather(idx[1], table_p).reshape(b // 2, l, 384)
    o0 = _tc_ln_half(w0, chars_embeddings, pos, g2, b2, 0, None)
    out = _tc_ln_half(w1, chars_embeddings, pos, g2, b2,
                      (b // 2) // RB, o0)
    return out
